# Initial kernel scaffold; baseline (speedup 1.0000x reference)
import jax
import jax.numpy as jnp
from jax.experimental import pallas as pl


def kernel(features, table, feature_offsets):
    raise NotImplementedError("write your pallas kernel here")


# speedup vs baseline: 3.1210x; 3.1210x over previous
import functools

import jax
import jax.numpy as jnp
from jax import lax
from jax.experimental import pallas as pl
from jax.experimental.pallas import tpu as pltpu
from jax.experimental.pallas import tpu_sc as plsc

B = 16384
F = 26
D = 128
NW = 32
PER_W = B * F // NW
CH = 128
NCH = PER_W // CH
NBUF = 2


def _sc_lookup(feats_hbm, offs_hbm, table_hbm, out_hbm,
               idx_v, offs_v, rows_v, gsem, ssem):
    wid = lax.axis_index("s") * 2 + lax.axis_index("c")

    pltpu.sync_copy(feats_hbm.at[wid], idx_v)
    pltpu.sync_copy(offs_hbm, offs_v)

    lane = lax.iota(jnp.int32, 16)

    def compute_body(j, _):
        for g in range(CH // 16):
            pos = j * CH + (g * 16) + lane
            off = plsc.load_gather(offs_v, [lax.rem(pos, F)])
            idx_v[j, pl.ds(g * 16, 16)] = idx_v[j, pl.ds(g * 16, 16)] + off
        return 0

    lax.fori_loop(0, NCH, compute_body, 0)

    def start_gather(j, slot):
        return pltpu.async_copy(
            table_hbm.at[idx_v.at[j]], rows_v.at[slot], gsem.at[slot])

    def start_scatter(j, slot):
        return pltpu.async_copy(
            rows_v.at[slot], out_hbm.at[wid, j], ssem.at[slot])

    start_gather(0, 0)

    def dma_body(j, _):
        slot = lax.rem(j, NBUF)
        nxt = lax.rem(j + 1, NBUF)
        pltpu.make_async_copy(
            table_hbm.at[idx_v.at[j]], rows_v.at[slot], gsem.at[slot]).wait()

        @pl.when(j + 1 < NCH)
        def _():
            start_gather(j + 1, nxt)

        start_scatter(j, slot)

        pltpu.make_async_copy(
            rows_v.at[slot], out_hbm.at[wid, j], ssem.at[slot]).wait()
        return 0

    lax.fori_loop(0, NCH, dma_body, 0)


@functools.partial(jax.jit, static_argnames=())
def _run(feats_flat, offs_pad, table):
    mesh = plsc.VectorSubcoreMesh(core_axis_name="c", subcore_axis_name="s")
    f = functools.partial(
        pl.kernel,
        out_type=jax.ShapeDtypeStruct((NW, NCH, CH, D), jnp.float32),
        mesh=mesh,
        scratch_types=[
            pltpu.VMEM((NCH, CH), jnp.int32),
            pltpu.VMEM((32,), jnp.int32),
            pltpu.VMEM((NBUF, CH, D), jnp.float32),
            pltpu.SemaphoreType.DMA((NBUF,)),
            pltpu.SemaphoreType.DMA((NBUF,)),
        ],
        compiler_params=pltpu.CompilerParams(needs_layout_passes=False),
    )(_sc_lookup)
    return f(feats_flat, offs_pad, table)


def kernel(features, table, feature_offsets):
    feats_flat = features.reshape(NW, NCH, CH)
    offs_pad = jnp.pad(feature_offsets, (0, 32 - F))
    out = _run(feats_flat, offs_pad, table)
    return out.reshape(B, F, D)


# speedup vs baseline: 3.7969x; 1.2166x over previous
import functools

import jax
import jax.numpy as jnp
from jax import lax
from jax.experimental import pallas as pl
from jax.experimental.pallas import tpu as pltpu
from jax.experimental.pallas import tpu_sc as plsc

B = 16384
F = 26
FP = 32
D = 128
NW = 32
FR_W = B // NW
PER_W = FR_W * F
FR_CH = 4
CH = FR_CH * FP
NCH = FR_W // FR_CH
NBUF = 2


def _sc_lookup(feats_hbm, offs_hbm, table_hbm, out_hbm,
               feats_v, idx_v, offs_v, rows_v, gsem, ssem):
    wid = lax.axis_index("s") * 2 + lax.axis_index("c")

    pltpu.sync_copy(feats_hbm.at[wid], feats_v)
    pltpu.sync_copy(offs_hbm, offs_v)

    lane = lax.iota(jnp.int32, 16)

    def compute_body(j, _):
        for g in range(CH // 16):
            k = g // 2
            r = (g % 2) * 16 + lane
            r_eff = jnp.minimum(r, F - 1)
            pos = (j * FR_CH + k) * F + r_eff
            feat = plsc.load_gather(feats_v, [pos])
            off = plsc.load_gather(offs_v, [r_eff])
            idx_v[j, pl.ds(g * 16, 16)] = feat + off
        return 0

    lax.fori_loop(0, NCH, compute_body, 0)

    def start_gather(j, slot):
        return pltpu.async_copy(
            table_hbm.at[idx_v.at[j]], rows_v.at[slot], gsem.at[slot])

    start_gather(0, 0)

    def dma_body(j, _):
        slot = lax.rem(j, NBUF)
        nxt = lax.rem(j + 1, NBUF)
        pltpu.make_async_copy(
            table_hbm.at[idx_v.at[j]], rows_v.at[slot], gsem.at[slot]).wait()

        @pl.when(j + 1 < NCH)
        def _():
            start_gather(j + 1, nxt)

        pltpu.async_copy(
            rows_v.at[slot], out_hbm.at[wid * NCH + j], ssem.at[slot])
        pltpu.make_async_copy(
            rows_v.at[slot], out_hbm.at[wid * NCH + j], ssem.at[slot]).wait()
        return 0

    lax.fori_loop(0, NCH, dma_body, 0)


@jax.jit
def _run(feats_flat, offs_pad, table):
    mesh = plsc.VectorSubcoreMesh(core_axis_name="c", subcore_axis_name="s")
    f = functools.partial(
        pl.kernel,
        out_type=jax.ShapeDtypeStruct((NW * NCH, CH, D), jnp.float32),
        mesh=mesh,
        scratch_types=[
            pltpu.VMEM((PER_W,), jnp.int32),
            pltpu.VMEM((NCH, CH), jnp.int32),
            pltpu.VMEM((32,), jnp.int32),
            pltpu.VMEM((NBUF, CH, D), jnp.float32),
            pltpu.SemaphoreType.DMA((NBUF,)),
            pltpu.SemaphoreType.DMA((NBUF,)),
        ],
        compiler_params=pltpu.CompilerParams(needs_layout_passes=False),
    )(_sc_lookup)
    return f(feats_flat, offs_pad, table)


def kernel(features, table, feature_offsets):
    feats_flat = features.reshape(NW, PER_W)
    offs_pad = jnp.pad(feature_offsets, (0, 32 - F))
    out = _run(feats_flat, offs_pad, table)
    return out.reshape(B, FP, D)[:, :F, :]


# speedup vs baseline: 3.8249x; 1.0074x over previous
import functools

import jax
import jax.numpy as jnp
from jax import lax
from jax.experimental import pallas as pl
from jax.experimental.pallas import tpu as pltpu
from jax.experimental.pallas import tpu_sc as plsc

B = 16384
F = 26
FP = 32
D = 128
NW = 32
FR_W = B // NW
PER_W = FR_W * F
FR_CH = 4
CH = FR_CH * FP
CHF = FR_CH * F
NCH = FR_W // FR_CH
NBUF = 3
NG = CH // 16


def _sc_lookup(feats_hbm, offs_hbm, table_hbm, out_hbm,
               feats_v, idx_v, offs_v, pat_v, rows_v, gsem, ssem):
    wid = lax.axis_index("s") * 2 + lax.axis_index("c")

    pltpu.sync_copy(feats_hbm.at[wid], feats_v)
    pltpu.sync_copy(offs_hbm, offs_v)

    lane = lax.iota(jnp.int32, 16)
    for g in range(NG):
        r_eff = jnp.minimum((g % 2) * 16 + lane, F - 1)
        pat_v[0, pl.ds(g * 16, 16)] = (g // 2) * F + r_eff
        pat_v[1, pl.ds(g * 16, 16)] = plsc.load_gather(offs_v, [r_eff])

    def compute_row(j):
        for g in range(NG):
            sl = pl.ds(g * 16, 16)
            feat = plsc.load_gather(feats_v, [j * CHF + pat_v[0, sl]])
            idx_v[j, sl] = feat + pat_v[1, sl]

    def start_gather(j, slot):
        return pltpu.async_copy(
            table_hbm.at[idx_v.at[j]], rows_v.at[slot], gsem.at[slot])

    def scatter_pair(j, slot):
        return (rows_v.at[slot], out_hbm.at[wid * NCH + j], ssem.at[slot])

    compute_row(0)
    start_gather(0, 0)
    compute_row(1)
    compute_row(2)

    def dma_body(j, _):
        slot = lax.rem(j, NBUF)
        nxt = lax.rem(j + 1, NBUF)

        pltpu.make_async_copy(
            table_hbm.at[idx_v.at[j]], rows_v.at[slot], gsem.at[slot]).wait()
        pltpu.async_copy(*scatter_pair(j, slot))

        @pl.when(j + 1 < NCH)
        def _():
            @pl.when(j + 1 >= NBUF)
            def _():
                pltpu.make_async_copy(*scatter_pair(j + 1 - NBUF, nxt)).wait()
            start_gather(j + 1, nxt)

        @pl.when(j + 3 < NCH)
        def _():
            compute_row(j + 3)
        return 0

    lax.fori_loop(0, NCH, dma_body, 0)

    for jj in range(NCH - NBUF + 1, NCH):
        pltpu.make_async_copy(*scatter_pair(jj, jj % NBUF)).wait()


@jax.jit
def _run(feats_flat, offs_pad, table):
    mesh = plsc.VectorSubcoreMesh(core_axis_name="c", subcore_axis_name="s")
    f = functools.partial(
        pl.kernel,
        out_type=jax.ShapeDtypeStruct((NW * NCH, CH, D), jnp.float32),
        mesh=mesh,
        scratch_types=[
            pltpu.VMEM((PER_W,), jnp.int32),
            pltpu.VMEM((NCH, CH), jnp.int32),
            pltpu.VMEM((32,), jnp.int32),
            pltpu.VMEM((2, CH), jnp.int32),
            pltpu.VMEM((NBUF, CH, D), jnp.float32),
            pltpu.SemaphoreType.DMA((NBUF,)),
            pltpu.SemaphoreType.DMA((NBUF,)),
        ],
        compiler_params=pltpu.CompilerParams(needs_layout_passes=False),
    )(_sc_lookup)
    return f(feats_flat, offs_pad, table)


def kernel(features, table, feature_offsets):
    feats_flat = features.reshape(NW, PER_W)
    offs_pad = jnp.pad(feature_offsets, (0, 32 - F))
    out = _run(feats_flat, offs_pad, table)
    return out.reshape(B, FP, D)[:, :F, :]


# speedup vs baseline: 4.7033x; 1.2297x over previous
import functools

import jax
import jax.numpy as jnp
from jax import lax
from jax.experimental import pallas as pl
from jax.experimental.pallas import tpu as pltpu
from jax.experimental.pallas import tpu_sc as plsc

B = 16384
F = 26
D = 128
NW = 32
FR_W = B // NW
PER_W = FR_W * F
FR_CH = 4
CHF = FR_CH * F
NCH = FR_W // FR_CH
NBUF = 3
STARTS = [0, 16, 32, 48, 64, 80, 88]


def _sc_lookup(feats_hbm, offs_hbm, table_hbm, out_hbm,
               feats_v, idx_v, offs_v, pat_v, rows_v, gsem, ssem):
    wid = lax.axis_index("s") * 2 + lax.axis_index("c")

    pltpu.sync_copy(feats_hbm.at[wid], feats_v)
    pltpu.sync_copy(offs_hbm, offs_v)

    lane = lax.iota(jnp.int32, 16)
    for s in STARTS:
        pat_v[pl.ds(s, 16)] = plsc.load_gather(offs_v, [lax.rem(s + lane, F)])

    def compute_row(j):
        for s in STARTS:
            sl = pl.ds(s, 16)
            feat = plsc.load_gather(feats_v, [j * CHF + s + lane])
            idx_v[j, sl] = feat + pat_v[sl]

    def start_gather(j, slot):
        return pltpu.async_copy(
            table_hbm.at[idx_v.at[j]], rows_v.at[slot], gsem.at[slot])

    def scatter_pairs(j, slot):
        return [(rows_v.at[slot, pl.ds(k * F, F)],
                 out_hbm.at[wid * NCH + j, k], ssem.at[slot])
                for k in range(FR_CH)]

    compute_row(0)
    start_gather(0, 0)
    compute_row(1)
    compute_row(2)

    def dma_body(j, _):
        slot = lax.rem(j, NBUF)
        nxt = lax.rem(j + 1, NBUF)

        pltpu.make_async_copy(
            table_hbm.at[idx_v.at[j]], rows_v.at[slot], gsem.at[slot]).wait()
        for p in scatter_pairs(j, slot):
            pltpu.async_copy(*p)

        @pl.when(j + 1 < NCH)
        def _():
            @pl.when(j + 1 >= NBUF)
            def _():
                for p in scatter_pairs(j + 1 - NBUF, nxt):
                    pltpu.make_async_copy(*p).wait()
            start_gather(j + 1, nxt)

        @pl.when(j + 3 < NCH)
        def _():
            compute_row(j + 3)
        return 0

    lax.fori_loop(0, NCH, dma_body, 0)

    for jj in range(NCH - NBUF + 1, NCH):
        for p in scatter_pairs(jj, jj % NBUF):
            pltpu.make_async_copy(*p).wait()


@jax.jit
def _run(feats_flat, offs_pad, table):
    mesh = plsc.VectorSubcoreMesh(core_axis_name="c", subcore_axis_name="s")
    f = functools.partial(
        pl.kernel,
        out_type=jax.ShapeDtypeStruct((NW * NCH, FR_CH, F, D), jnp.float32),
        mesh=mesh,
        scratch_types=[
            pltpu.VMEM((PER_W,), jnp.int32),
            pltpu.VMEM((NCH, CHF), jnp.int32),
            pltpu.VMEM((128,), jnp.int32),
            pltpu.VMEM((CHF,), jnp.int32),
            pltpu.VMEM((NBUF, CHF, D), jnp.float32),
            pltpu.SemaphoreType.DMA((NBUF,)),
            pltpu.SemaphoreType.DMA((NBUF,)),
        ],
        compiler_params=pltpu.CompilerParams(needs_layout_passes=False),
    )(_sc_lookup)
    return f(feats_flat, offs_pad, table)


def kernel(features, table, feature_offsets):
    feats_flat = features.reshape(NW, PER_W)
    offs_pad = jnp.pad(feature_offsets, (0, 128 - F))
    out = _run(feats_flat, offs_pad, table)
    return out.reshape(B, F, D)
